# baseline (speedup 1.0000x reference)
import functools

import jax
import jax.numpy as jnp
from jax import lax
from jax.experimental import pallas as pl
from jax.experimental.pallas import tpu as pltpu
from jax.experimental.pallas import tpu_sc as plsc

NUM_CORES = 2
NUM_SUBCORES = 16
LANES = 16
NUM_WORKERS = NUM_CORES * NUM_SUBCORES
IDX_CHUNK = 128


def _make_kernel(batch, dim):
    assert batch % (NUM_WORKERS * IDX_CHUNK) == 0
    assert dim % LANES == 0
    b_per_w = batch // NUM_WORKERS
    chunks = b_per_w // IDX_CHUNK
    mesh = plsc.VectorSubcoreMesh(
        core_axis_name="c", subcore_axis_name="s",
        num_cores=NUM_CORES, num_subcores=NUM_SUBCORES)

    @functools.partial(
        pl.kernel,
        out_type=jax.ShapeDtypeStruct((batch, dim), jnp.float32),
        mesh=mesh,
        compiler_params=pltpu.CompilerParams(use_tc_tiling_on_sc=False),
        scratch_types=[
            pltpu.VMEM((chunks, IDX_CHUNK), jnp.int32),
            pltpu.VMEM((chunks, IDX_CHUNK), jnp.int32),
            pltpu.VMEM((b_per_w, dim), jnp.float32),
            pltpu.VMEM((b_per_w, dim), jnp.float32),
            pltpu.SemaphoreType.DMA,
        ],
    )
    def gmf_kernel(uids_hbm, iids_hbm, uemb_hbm, iemb_hbm, out_hbm,
                   uidx_v, iidx_v, urows_v, irows_v, sem):
        wid = lax.axis_index("s") * NUM_CORES + lax.axis_index("c")
        row_base = wid * chunks
        pltpu.sync_copy(uids_hbm.at[pl.ds(row_base, chunks), :], uidx_v)
        pltpu.sync_copy(iids_hbm.at[pl.ds(row_base, chunks), :], iidx_v)
        copies = []
        for j in range(chunks):
            dst = pl.ds(j * IDX_CHUNK, IDX_CHUNK)
            copies.append(pltpu.async_copy(
                uemb_hbm.at[uidx_v.at[j]], urows_v.at[dst, :], sem))
            copies.append(pltpu.async_copy(
                iemb_hbm.at[iidx_v.at[j]], irows_v.at[dst, :], sem))
        for c in copies:
            c.wait()

        @pl.loop(0, b_per_w)
        def _mul(b):
            for h in range(dim // LANES):
                sl = pl.ds(h * LANES, LANES)
                urows_v[b, sl] = urows_v[b, sl] * irows_v[b, sl]

        pltpu.sync_copy(
            urows_v, out_hbm.at[pl.ds(wid * b_per_w, b_per_w), :])

    return gmf_kernel


def kernel(user_ids, item_ids, user_emb, item_emb):
    batch = user_ids.shape[0]
    dim = user_emb.shape[1]
    uids = user_ids.astype(jnp.int32).reshape(-1, IDX_CHUNK)
    iids = item_ids.astype(jnp.int32).reshape(-1, IDX_CHUNK)
    return _make_kernel(batch, dim)(uids, iids, user_emb, item_emb)


# speedup vs baseline: 3.2421x; 3.2421x over previous
import functools

import jax
import jax.numpy as jnp
from jax import lax
from jax.experimental import pallas as pl
from jax.experimental.pallas import tpu as pltpu
from jax.experimental.pallas import tpu_sc as plsc

NUM_CORES = 2
NUM_SUBCORES = 16
LANES = 16
NUM_WORKERS = NUM_CORES * NUM_SUBCORES
GROUP = 16


def _make_kernel(batch, dim, vocab):
    assert batch % (NUM_WORKERS * 2) == 0
    b_per_w = batch // NUM_WORKERS
    assert b_per_w % GROUP == 0
    n_groups = b_per_w // GROUP
    mesh = plsc.VectorSubcoreMesh(
        core_axis_name="c", subcore_axis_name="s",
        num_cores=NUM_CORES, num_subcores=NUM_SUBCORES)

    @functools.partial(
        pl.kernel,
        out_type=jax.ShapeDtypeStruct((dim, batch), jnp.float32),
        mesh=mesh,
        compiler_params=pltpu.CompilerParams(
            use_tc_tiling_on_sc=True, needs_layout_passes=False),
        scratch_types=[
            pltpu.VMEM((2 * b_per_w,), jnp.int32),
            pltpu.VMEM((GROUP, dim, 128), jnp.float32),
            pltpu.VMEM((dim, b_per_w), jnp.float32),
            pltpu.SemaphoreType.DMA,
        ],
    )
    def gmf_kernel(uids_hbm, iids_hbm, uembt_hbm, iembt_hbm, outt_hbm,
                   ids_v, blocks_v, out_v, sem):
        wid = lax.axis_index("s") * NUM_CORES + lax.axis_index("c")
        pair = wid // 2
        off = (wid % 2) * b_per_w
        fvec = jnp.arange(LANES, dtype=jnp.int32)

        def one_pass(ids_hbm, emb_hbm, first):
            pltpu.sync_copy(ids_hbm.at[pl.ds(pair * 2 * b_per_w, 2 * b_per_w)],
                            ids_v)

            @pl.loop(0, n_groups)
            def _group(g):
                b0 = g * GROUP
                chunk = ids_v[pl.ds(off + b0, GROUP)]
                lanes = chunk & 127
                for j in range(GROUP):
                    u = chunk[j]
                    blk = pl.multiple_of((u >> 7) * 128, 128)
                    pltpu.async_copy(
                        emb_hbm.at[:, pl.ds(blk, 128)], blocks_v.at[j], sem)
                for j in range(GROUP):
                    pltpu.make_async_copy(
                        emb_hbm.at[:, pl.ds(0, 128)], blocks_v.at[j], sem
                    ).wait()
                for j in range(GROUP):
                    b = b0 + j
                    lane = jnp.full((LANES,), lanes[j], jnp.int32)
                    jv = jnp.full((LANES,), j, jnp.int32)
                    bv = jnp.full((LANES,), b, jnp.int32)
                    for h in range(dim // LANES):
                        fv = fvec + h * LANES
                        val = plsc.load_gather(blocks_v, [jv, fv, lane])
                        if not first:
                            val = val * plsc.load_gather(out_v, [fv, bv])
                        plsc.store_scatter(out_v, [fv, bv], val)

        one_pass(uids_hbm, uembt_hbm, True)
        one_pass(iids_hbm, iembt_hbm, False)
        pltpu.sync_copy(out_v, outt_hbm.at[:, pl.ds(wid * b_per_w, b_per_w)])

    return gmf_kernel


def kernel(user_ids, item_ids, user_emb, item_emb):
    batch = user_ids.shape[0]
    vocab, dim = user_emb.shape
    uids = user_ids.astype(jnp.int32)
    iids = item_ids.astype(jnp.int32)
    outt = _make_kernel(batch, dim, vocab)(uids, iids, user_emb.T, item_emb.T)
    return outt.T
